# baseline (speedup 1.0000x reference)
import functools

import jax
import jax.numpy as jnp
from jax import lax
from jax.experimental import pallas as pl
from jax.experimental.pallas import tpu as pltpu
from jax.experimental.pallas import tpu_sc as plsc

_K = 3
_INV_TEMP = 100.0


def _splat(vec, t, L):
    idx = jnp.full((L,), t, jnp.int32)
    dnums = lax.GatherDimensionNumbers(
        offset_dims=(), collapsed_slice_dims=(0,), start_index_map=(0,))
    return lax.gather(vec, idx[:, None], dnums, (1,),
                      mode=lax.GatherScatterMode.PROMISE_IN_BOUNDS)


def _make_topk_body(TN, TCB, C, nj):
    def body(x_ref, cb_ref, w_ref, idx_ref, vals_s, idx_s):
        j = pl.program_id(1)

        @pl.when(j == 0)
        def _init():
            vals_s[...] = jnp.full(vals_s.shape, -jnp.inf, jnp.float32)
            idx_s[...] = jnp.zeros(idx_s.shape, jnp.int32)

        x = x_ref[...]
        cb = cb_ref[...]
        s = lax.dot_general(x, cb, (((1,), (1,)), ((), ())),
                            preferred_element_type=jnp.float32)


        v1 = vals_s[:, 0:1]
        v2 = vals_s[:, 1:2]
        v3 = vals_s[:, 2:3]
        i1 = idx_s[:, 0:1]
        i2 = idx_s[:, 1:2]
        i3 = idx_s[:, 2:3]

        NEG = jnp.float32(-1e30)
        col = jax.lax.broadcasted_iota(jnp.int32, (TN, TCB), 1)
        BIG = jnp.int32(2 ** 30)
        msks = []
        ms = []
        for _ in range(_K):
            m = jnp.max(s, axis=1, keepdims=True)
            msk = s >= m
            msks.append(msk)
            ms.append(m)
            s = jnp.where(msk, NEG, s)
        SH = (TCB - 1).bit_length()
        packed = (jnp.where(msks[0], col, 0)
                  + jnp.where(msks[1], col << SH, 0))
        psum = jnp.sum(packed, axis=1, keepdims=True)
        e1 = jnp.minimum(psum & ((1 << SH) - 1), TCB - 1) + j * TCB
        e2 = jnp.minimum(psum >> SH, TCB - 1) + j * TCB
        e3 = (jnp.min(jnp.where(msks[2], col, BIG), axis=1, keepdims=True)
              + j * TCB)
        ems = [(ms[0], e1), (ms[1], e2), (ms[2], e3)]
        for m, e in ems:
            b1 = m > v1
            nv1 = jnp.where(b1, m, v1)
            ni1 = jnp.where(b1, e, i1)
            c = jnp.where(b1, v1, m)
            ci = jnp.where(b1, i1, e)
            b2 = c > v2
            nv2 = jnp.where(b2, c, v2)
            ni2 = jnp.where(b2, ci, i2)
            c2 = jnp.where(b2, v2, c)
            ci2 = jnp.where(b2, i2, ci)
            b3 = c2 > v3
            v3 = jnp.where(b3, c2, v3)
            i3 = jnp.where(b3, ci2, i3)
            v1, v2, i1, i2 = nv1, nv2, ni1, ni2

        vals_s[:, 0:1] = v1
        vals_s[:, 1:2] = v2
        vals_s[:, 2:3] = v3
        idx_s[:, 0:1] = i1
        idx_s[:, 1:2] = i2
        idx_s[:, 2:3] = i3

        @pl.when(j == nj - 1)
        def _finish():
            mx = jnp.maximum(v1 * _INV_TEMP, 0.0)
            e1 = jnp.exp(v1 * _INV_TEMP - mx)
            e2 = jnp.exp(v2 * _INV_TEMP - mx)
            e3 = jnp.exp(v3 * _INV_TEMP - mx)
            ebg = jnp.exp(-mx)
            denom = e1 + e2 + e3 + jnp.float32(C - _K) * ebg
            inv = 1.0 / denom
            pbg = ebg * inv
            w1 = e1 * inv - pbg
            w2 = e2 * inv - pbg
            w3 = e3 * inv - pbg
            zero = jnp.zeros((TN, 1), jnp.float32)
            w_ref[...] = jnp.concatenate(
                [w1, w2, w3, pbg, zero, zero, zero, zero], axis=1)
            izero = jnp.zeros((TN, 1), jnp.int32)
            idx_ref[...] = jnp.concatenate(
                [i1, i2, i3, izero, izero, izero, izero, izero], axis=1)

    return body


def _csum_body(cb_ref, csum_ref):
    j = pl.program_id(0)

    @pl.when(j == 0)
    def _init():
        csum_ref[...] = jnp.zeros(csum_ref.shape, jnp.float32)

    csum_ref[0:1, :] = csum_ref[0:1, :] + jnp.sum(
        cb_ref[...], axis=0, keepdims=True)


def _csum(codebook, TCB):
    C, D = codebook.shape
    return pl.pallas_call(
        _csum_body,
        grid=(C // TCB,),
        in_specs=[pl.BlockSpec((TCB, D), lambda j: (j, 0))],
        out_specs=pl.BlockSpec((8, D), lambda j: (0, 0)),
        out_shape=jax.ShapeDtypeStruct((8, D), jnp.float32),
    )(codebook)


def _stage1(x, codebook, TN, TCB):
    N, D = x.shape
    C = codebook.shape[0]
    ni = N // TN
    nj = C // TCB
    body = _make_topk_body(TN, TCB, C, nj)
    w8, idx8 = pl.pallas_call(
        body,
        grid=(ni, nj),
        in_specs=[
            pl.BlockSpec((TN, D), lambda i, j: (i, 0)),
            pl.BlockSpec((TCB, D), lambda i, j: (j, 0)),
        ],
        out_specs=[
            pl.BlockSpec((TN, 8), lambda i, j: (i, 0)),
            pl.BlockSpec((TN, 8), lambda i, j: (i, 0)),
        ],
        out_shape=[
            jax.ShapeDtypeStruct((N, 8), jnp.float32),
            jax.ShapeDtypeStruct((N, 8), jnp.int32),
        ],
        scratch_shapes=[
            pltpu.VMEM((TN, 8), jnp.float32),
            pltpu.VMEM((TN, 8), jnp.int32),
        ],
        compiler_params=pltpu.CompilerParams(
            dimension_semantics=("arbitrary", "arbitrary")),
    )(x, codebook)
    return w8, idx8


def _stage2_sc(codebook, i1, i2, i3, w1, w2, w3, pbg, csum):
    N = i1.shape[0]
    C, D = codebook.shape
    info = plsc.get_sparse_core_info()
    NW = info.num_cores * info.num_subcores
    L = info.num_lanes
    TPW = N // NW
    CH = 16
    NCH = TPW // CH
    DL = D // L
    mesh = plsc.VectorSubcoreMesh(core_axis_name="c", subcore_axis_name="s")

    NCH2 = NCH // 2

    @functools.partial(
        pl.kernel,
        mesh=mesh,
        out_type=jax.ShapeDtypeStruct((N, D), jnp.float32),
        scratch_types=[
            pltpu.VMEM((TPW,), jnp.int32),
            pltpu.VMEM((TPW,), jnp.int32),
            pltpu.VMEM((TPW,), jnp.int32),
            pltpu.VMEM((TPW,), jnp.float32),
            pltpu.VMEM((TPW,), jnp.float32),
            pltpu.VMEM((TPW,), jnp.float32),
            pltpu.VMEM((TPW,), jnp.float32),
            pltpu.VMEM((D,), jnp.float32),
            pltpu.VMEM((2, _K * CH, D), jnp.float32),
            pltpu.VMEM((CH, D), jnp.float32),
            pltpu.SemaphoreType.DMA,
            pltpu.SemaphoreType.DMA,
        ],
    )
    def k(cb_h, i1_h, i2_h, i3_h, w1_h, w2_h, w3_h, pbg_h, csum_h, out_h,
          i1_v, i2_v, i3_v, w1_v, w2_v, w3_v, pbg_v, csum_v, rows_v, out_v,
          sem0, sem1):
        wid = lax.axis_index("s") * info.num_cores + lax.axis_index("c")
        base = pl.multiple_of(wid * TPW, TPW)
        pltpu.sync_copy(i1_h.at[pl.ds(base, TPW)], i1_v)
        pltpu.sync_copy(i2_h.at[pl.ds(base, TPW)], i2_v)
        pltpu.sync_copy(i3_h.at[pl.ds(base, TPW)], i3_v)
        pltpu.sync_copy(w1_h.at[pl.ds(base, TPW)], w1_v)
        pltpu.sync_copy(w2_h.at[pl.ds(base, TPW)], w2_v)
        pltpu.sync_copy(w3_h.at[pl.ds(base, TPW)], w3_v)
        pltpu.sync_copy(pbg_h.at[pl.ds(base, TPW)], pbg_v)
        pltpu.sync_copy(csum_h, csum_v)
        sems = (sem0, sem1)

        def issue(c, p):
            off = pl.multiple_of(c * CH, CH)
            pltpu.async_copy(cb_h.at[i1_v.at[pl.ds(off, CH)]],
                             rows_v.at[p, pl.ds(0, CH)], sems[p])
            pltpu.async_copy(cb_h.at[i2_v.at[pl.ds(off, CH)]],
                             rows_v.at[p, pl.ds(CH, CH)], sems[p])
            pltpu.async_copy(cb_h.at[i3_v.at[pl.ds(off, CH)]],
                             rows_v.at[p, pl.ds(2 * CH, CH)], sems[p])

        def drain(p):
            pltpu.make_async_copy(cb_h.at[pl.ds(0, _K * CH)],
                                  rows_v.at[p], sems[p]).wait()

        def compute(c, p):
            off = pl.multiple_of(c * CH, CH)
            w1c = w1_v[pl.ds(off, CH)]
            w2c = w2_v[pl.ds(off, CH)]
            w3c = w3_v[pl.ds(off, CH)]
            pbgc = pbg_v[pl.ds(off, CH)]
            scs = [csum_v[pl.ds(d * L, L)] for d in range(DL)]
            for t in range(CH):
                s1 = _splat(w1c, t, L)
                s2 = _splat(w2c, t, L)
                s3 = _splat(w3c, t, L)
                sb = _splat(pbgc, t, L)
                for d in range(DL):
                    dsl = pl.ds(d * L, L)
                    val = (s1 * rows_v[p, t, dsl]
                           + s2 * rows_v[p, CH + t, dsl]
                           + s3 * rows_v[p, 2 * CH + t, dsl] + sb * scs[d])
                    out_v[t, dsl] = val
            pltpu.sync_copy(out_v, out_h.at[pl.ds(base + off, CH)])

        issue(0, 0)

        def pair(c2, carry):
            c = pl.multiple_of(c2 * 2, 2)

            @pl.when(c + 1 < NCH)
            def _():
                issue(c + 1, 1)

            drain(0)
            compute(c, 0)

            @pl.when(c + 2 < NCH)
            def _():
                issue(c + 2, 0)

            drain(1)
            compute(c + 1, 1)
            return carry

        lax.fori_loop(0, NCH2, pair, 0)

    return k(codebook, i1, i2, i3, w1, w2, w3, pbg, csum)


def kernel(x, codebook):
    N, D = x.shape
    TN = min(512, N)
    TCB = min(8192, codebook.shape[0])
    csum8 = _csum(codebook, 2048)
    HALF = N // 2
    zs = []
    for h in range(2):
        xs = lax.slice_in_dim(x, h * HALF, (h + 1) * HALF, axis=0)
        w8, idx8 = _stage1(xs, codebook, TN, TCB)
        zs.append(_stage2_sc(
            codebook,
            idx8[:, 0], idx8[:, 1], idx8[:, 2],
            w8[:, 0], w8[:, 1], w8[:, 2], w8[:, 3],
            csum8[0],
        ))
    return jnp.concatenate(zs, axis=0)


# speedup vs baseline: 1.0531x; 1.0531x over previous
import functools

import jax
import jax.numpy as jnp
from jax import lax
from jax.experimental import pallas as pl
from jax.experimental.pallas import tpu as pltpu
from jax.experimental.pallas import tpu_sc as plsc

_K = 3
_INV_TEMP = 100.0


def _splat(vec, t, L):
    idx = jnp.full((L,), t, jnp.int32)
    dnums = lax.GatherDimensionNumbers(
        offset_dims=(), collapsed_slice_dims=(0,), start_index_map=(0,))
    return lax.gather(vec, idx[:, None], dnums, (1,),
                      mode=lax.GatherScatterMode.PROMISE_IN_BOUNDS)


def _make_topk_body(TN, TCB, C, nj):
    def body(x_ref, cb_ref, w_ref, idx_ref, vals_s, idx_s):
        j = pl.program_id(1)

        @pl.when(j == 0)
        def _init():
            vals_s[...] = jnp.full(vals_s.shape, -jnp.inf, jnp.float32)
            idx_s[...] = jnp.zeros(idx_s.shape, jnp.int32)

        x = x_ref[...]
        cb = cb_ref[...]
        s = lax.dot_general(x, cb, (((1,), (1,)), ((), ())),
                            preferred_element_type=jnp.float32)


        v1 = vals_s[:, 0:1]
        v2 = vals_s[:, 1:2]
        v3 = vals_s[:, 2:3]
        i1 = idx_s[:, 0:1]
        i2 = idx_s[:, 1:2]
        i3 = idx_s[:, 2:3]

        NEG = jnp.float32(-1e30)
        col = jax.lax.broadcasted_iota(jnp.int32, (TN, TCB), 1)
        BIG = jnp.int32(2 ** 30)
        msks = []
        ms = []
        for _ in range(_K):
            m = jnp.max(s, axis=1, keepdims=True)
            msk = s >= m
            msks.append(msk)
            ms.append(m)
            s = jnp.where(msk, NEG, s)
        SH = (TCB - 1).bit_length()
        packed = (jnp.where(msks[0], col, 0)
                  + jnp.where(msks[1], col << SH, 0))
        psum = jnp.sum(packed, axis=1, keepdims=True)
        e1 = jnp.minimum(psum & ((1 << SH) - 1), TCB - 1) + j * TCB
        e2 = jnp.minimum(psum >> SH, TCB - 1) + j * TCB
        e3 = (jnp.minimum(jnp.sum(jnp.where(msks[2], col, 0), axis=1,
                                  keepdims=True), TCB - 1) + j * TCB)
        ems = [(ms[0], e1), (ms[1], e2), (ms[2], e3)]
        for m, e in ems:
            b1 = m > v1
            nv1 = jnp.where(b1, m, v1)
            ni1 = jnp.where(b1, e, i1)
            c = jnp.where(b1, v1, m)
            ci = jnp.where(b1, i1, e)
            b2 = c > v2
            nv2 = jnp.where(b2, c, v2)
            ni2 = jnp.where(b2, ci, i2)
            c2 = jnp.where(b2, v2, c)
            ci2 = jnp.where(b2, i2, ci)
            b3 = c2 > v3
            v3 = jnp.where(b3, c2, v3)
            i3 = jnp.where(b3, ci2, i3)
            v1, v2, i1, i2 = nv1, nv2, ni1, ni2

        vals_s[:, 0:1] = v1
        vals_s[:, 1:2] = v2
        vals_s[:, 2:3] = v3
        idx_s[:, 0:1] = i1
        idx_s[:, 1:2] = i2
        idx_s[:, 2:3] = i3

        @pl.when(j == nj - 1)
        def _finish():
            mx = jnp.maximum(v1 * _INV_TEMP, 0.0)
            e1 = jnp.exp(v1 * _INV_TEMP - mx)
            e2 = jnp.exp(v2 * _INV_TEMP - mx)
            e3 = jnp.exp(v3 * _INV_TEMP - mx)
            ebg = jnp.exp(-mx)
            denom = e1 + e2 + e3 + jnp.float32(C - _K) * ebg
            inv = 1.0 / denom
            pbg = ebg * inv
            w1 = e1 * inv - pbg
            w2 = e2 * inv - pbg
            w3 = e3 * inv - pbg
            zero = jnp.zeros((TN, 1), jnp.float32)
            w_ref[...] = jnp.concatenate(
                [w1, w2, w3, pbg, zero, zero, zero, zero], axis=1)
            izero = jnp.zeros((TN, 1), jnp.int32)
            idx_ref[...] = jnp.concatenate(
                [i1, i2, i3, izero, izero, izero, izero, izero], axis=1)

    return body


def _csum_body(cb_ref, csum_ref):
    j = pl.program_id(0)

    @pl.when(j == 0)
    def _init():
        csum_ref[...] = jnp.zeros(csum_ref.shape, jnp.float32)

    csum_ref[0:1, :] = csum_ref[0:1, :] + jnp.sum(
        cb_ref[...], axis=0, keepdims=True)


def _csum(codebook, TCB):
    C, D = codebook.shape
    return pl.pallas_call(
        _csum_body,
        grid=(C // TCB,),
        in_specs=[pl.BlockSpec((TCB, D), lambda j: (j, 0))],
        out_specs=pl.BlockSpec((8, D), lambda j: (0, 0)),
        out_shape=jax.ShapeDtypeStruct((8, D), jnp.float32),
    )(codebook)


def _stage1(x, codebook, TN, TCB):
    N, D = x.shape
    C = codebook.shape[0]
    ni = N // TN
    nj = C // TCB
    body = _make_topk_body(TN, TCB, C, nj)
    w8, idx8 = pl.pallas_call(
        body,
        grid=(ni, nj),
        in_specs=[
            pl.BlockSpec((TN, D), lambda i, j: (i, 0)),
            pl.BlockSpec((TCB, D), lambda i, j: (j, 0)),
        ],
        out_specs=[
            pl.BlockSpec((TN, 8), lambda i, j: (i, 0)),
            pl.BlockSpec((TN, 8), lambda i, j: (i, 0)),
        ],
        out_shape=[
            jax.ShapeDtypeStruct((N, 8), jnp.float32),
            jax.ShapeDtypeStruct((N, 8), jnp.int32),
        ],
        scratch_shapes=[
            pltpu.VMEM((TN, 8), jnp.float32),
            pltpu.VMEM((TN, 8), jnp.int32),
        ],
        compiler_params=pltpu.CompilerParams(
            dimension_semantics=("arbitrary", "arbitrary")),
    )(x, codebook)
    return w8, idx8


def _stage2_sc(codebook, i1, i2, i3, w1, w2, w3, pbg, csum):
    N = i1.shape[0]
    C, D = codebook.shape
    info = plsc.get_sparse_core_info()
    NW = info.num_cores * info.num_subcores
    L = info.num_lanes
    TPW = N // NW
    CH = 16
    NCH = TPW // CH
    DL = D // L
    mesh = plsc.VectorSubcoreMesh(core_axis_name="c", subcore_axis_name="s")

    NCH2 = NCH // 2

    @functools.partial(
        pl.kernel,
        mesh=mesh,
        out_type=jax.ShapeDtypeStruct((N, D), jnp.float32),
        scratch_types=[
            pltpu.VMEM((TPW,), jnp.int32),
            pltpu.VMEM((TPW,), jnp.int32),
            pltpu.VMEM((TPW,), jnp.int32),
            pltpu.VMEM((TPW,), jnp.float32),
            pltpu.VMEM((TPW,), jnp.float32),
            pltpu.VMEM((TPW,), jnp.float32),
            pltpu.VMEM((TPW,), jnp.float32),
            pltpu.VMEM((D,), jnp.float32),
            pltpu.VMEM((2, _K * CH, D), jnp.float32),
            pltpu.VMEM((CH, D), jnp.float32),
            pltpu.SemaphoreType.DMA,
            pltpu.SemaphoreType.DMA,
        ],
    )
    def k(cb_h, i1_h, i2_h, i3_h, w1_h, w2_h, w3_h, pbg_h, csum_h, out_h,
          i1_v, i2_v, i3_v, w1_v, w2_v, w3_v, pbg_v, csum_v, rows_v, out_v,
          sem0, sem1):
        wid = lax.axis_index("s") * info.num_cores + lax.axis_index("c")
        base = pl.multiple_of(wid * TPW, TPW)
        pltpu.sync_copy(i1_h.at[pl.ds(base, TPW)], i1_v)
        pltpu.sync_copy(i2_h.at[pl.ds(base, TPW)], i2_v)
        pltpu.sync_copy(i3_h.at[pl.ds(base, TPW)], i3_v)
        pltpu.sync_copy(w1_h.at[pl.ds(base, TPW)], w1_v)
        pltpu.sync_copy(w2_h.at[pl.ds(base, TPW)], w2_v)
        pltpu.sync_copy(w3_h.at[pl.ds(base, TPW)], w3_v)
        pltpu.sync_copy(pbg_h.at[pl.ds(base, TPW)], pbg_v)
        pltpu.sync_copy(csum_h, csum_v)
        sems = (sem0, sem1)

        def issue(c, p):
            off = pl.multiple_of(c * CH, CH)
            pltpu.async_copy(cb_h.at[i1_v.at[pl.ds(off, CH)]],
                             rows_v.at[p, pl.ds(0, CH)], sems[p])
            pltpu.async_copy(cb_h.at[i2_v.at[pl.ds(off, CH)]],
                             rows_v.at[p, pl.ds(CH, CH)], sems[p])
            pltpu.async_copy(cb_h.at[i3_v.at[pl.ds(off, CH)]],
                             rows_v.at[p, pl.ds(2 * CH, CH)], sems[p])

        def drain(p):
            pltpu.make_async_copy(cb_h.at[pl.ds(0, _K * CH)],
                                  rows_v.at[p], sems[p]).wait()

        def compute(c, p):
            off = pl.multiple_of(c * CH, CH)
            w1c = w1_v[pl.ds(off, CH)]
            w2c = w2_v[pl.ds(off, CH)]
            w3c = w3_v[pl.ds(off, CH)]
            pbgc = pbg_v[pl.ds(off, CH)]
            scs = [csum_v[pl.ds(d * L, L)] for d in range(DL)]
            for t in range(CH):
                s1 = _splat(w1c, t, L)
                s2 = _splat(w2c, t, L)
                s3 = _splat(w3c, t, L)
                sb = _splat(pbgc, t, L)
                for d in range(DL):
                    dsl = pl.ds(d * L, L)
                    val = (s1 * rows_v[p, t, dsl]
                           + s2 * rows_v[p, CH + t, dsl]
                           + s3 * rows_v[p, 2 * CH + t, dsl] + sb * scs[d])
                    out_v[t, dsl] = val
            pltpu.sync_copy(out_v, out_h.at[pl.ds(base + off, CH)])

        issue(0, 0)

        def pair(c2, carry):
            c = pl.multiple_of(c2 * 2, 2)

            @pl.when(c + 1 < NCH)
            def _():
                issue(c + 1, 1)

            drain(0)
            compute(c, 0)

            @pl.when(c + 2 < NCH)
            def _():
                issue(c + 2, 0)

            drain(1)
            compute(c + 1, 1)
            return carry

        lax.fori_loop(0, NCH2, pair, 0)

    return k(codebook, i1, i2, i3, w1, w2, w3, pbg, csum)


def kernel(x, codebook):
    N, D = x.shape
    TN = min(512, N)
    TCB = min(8192, codebook.shape[0])
    csum8 = _csum(codebook, 2048)
    HALF = N // 2
    zs = []
    for h in range(2):
        xs = lax.slice_in_dim(x, h * HALF, (h + 1) * HALF, axis=0)
        w8, idx8 = _stage1(xs, codebook, TN, TCB)
        zs.append(_stage2_sc(
            codebook,
            idx8[:, 0], idx8[:, 1], idx8[:, 2],
            w8[:, 0], w8[:, 1], w8[:, 2], w8[:, 3],
            csum8[0],
        ))
    return jnp.concatenate(zs, axis=0)
